# initial kernel scaffold (unmeasured)
import jax
import jax.numpy as jnp
from jax import lax
from jax.experimental import pallas as pl
from jax.experimental.pallas import tpu as pltpu

N_DEV = 4


def kernel(x, w_mat, scale_x, scale_w):
    x8 = x.astype(jnp.float8_e5m2)
    w8 = w_mat.astype(jnp.float8_e5m2)
    m_per, k = x8.shape
    _, n_per = w8.shape

    def body(x_ref, w_ref, sx_ref, sw_ref, out_ref, comm_ref, send_sems, recv_sems):
        my = lax.axis_index("i")
        left = lax.rem(my + N_DEV - 1, N_DEV)
        right = lax.rem(my + 1, N_DEV)

        barrier_sem = pltpu.get_barrier_semaphore()
        for nbr in (left, right):
            pl.semaphore_signal(
                barrier_sem, inc=1,
                device_id=(nbr,), device_id_type=pl.DeviceIdType.MESH,
            )
        pl.semaphore_wait(barrier_sem, 2)

        scale = sx_ref[0] * sw_ref[0]

        out_ref[pl.ds(my * m_per, m_per), :] = (
            jnp.dot(x_ref[...], w_ref[...], preferred_element_type=jnp.float32)
            * scale
        )

        for h in range(N_DEV - 1):
            src = x_ref if h == 0 else comm_ref.at[h - 1]
            rdma = pltpu.make_async_remote_copy(
                src_ref=src,
                dst_ref=comm_ref.at[h],
                send_sem=send_sems.at[h],
                recv_sem=recv_sems.at[h],
                device_id=(right,),
                device_id_type=pl.DeviceIdType.MESH,
            )
            rdma.start()
            rdma.wait()
            origin = lax.rem(my + N_DEV - 1 - h, N_DEV)
            out_ref[pl.ds(origin * m_per, m_per), :] = (
                jnp.dot(comm_ref[h], w_ref[...], preferred_element_type=jnp.float32)
                * scale
            )

    return pl.pallas_call(
        body,
        out_shape=jax.ShapeDtypeStruct((N_DEV * m_per, n_per), jnp.float32),
        in_specs=[
            pl.BlockSpec(memory_space=pltpu.VMEM),
            pl.BlockSpec(memory_space=pltpu.VMEM),
            pl.BlockSpec(memory_space=pltpu.SMEM),
            pl.BlockSpec(memory_space=pltpu.SMEM),
        ],
        out_specs=pl.BlockSpec(memory_space=pltpu.VMEM),
        scratch_shapes=[
            pltpu.VMEM((N_DEV - 1, m_per, k), jnp.float8_e5m2),
            pltpu.SemaphoreType.DMA((N_DEV - 1,)),
            pltpu.SemaphoreType.DMA((N_DEV - 1,)),
        ],
        compiler_params=pltpu.CompilerParams(collective_id=0),
    )(x8, w8, scale_x, scale_w)


# baseline (device time: 241565 ns/iter reference)
import jax
import jax.numpy as jnp
from jax import lax
from jax.experimental import pallas as pl
from jax.experimental.pallas import tpu as pltpu

N_DEV = 4


def kernel(x, w_mat, scale_x, scale_w):
    x8 = x.astype(jnp.float8_e5m2)
    w8 = w_mat.astype(jnp.float8_e5m2)
    m_per, k = x8.shape
    _, n_per = w8.shape

    def body(x_ref, w_ref, sx_ref, sw_ref, out_ref, comm_ref, send_sems, recv_sems):
        my = lax.axis_index("i")
        left = lax.rem(my + N_DEV - 1, N_DEV)
        right = lax.rem(my + 1, N_DEV)

        barrier_sem = pltpu.get_barrier_semaphore()
        for nbr in (left, right):
            pl.semaphore_signal(
                barrier_sem, inc=1,
                device_id=(nbr,), device_id_type=pl.DeviceIdType.MESH,
            )
        pl.semaphore_wait(barrier_sem, 2)

        scale = sx_ref[0] * sw_ref[0]

        out_ref[pl.ds(my * m_per, m_per), :] = (
            jnp.dot(x_ref[...], w_ref[...], preferred_element_type=jnp.float32)
            * scale
        )

        for h in range(N_DEV - 1):
            src = x_ref if h == 0 else comm_ref.at[h - 1]
            rdma = pltpu.make_async_remote_copy(
                src_ref=src,
                dst_ref=comm_ref.at[h],
                send_sem=send_sems.at[h],
                recv_sem=recv_sems.at[h],
                device_id=(right,),
                device_id_type=pl.DeviceIdType.MESH,
            )
            rdma.start()
            rdma.wait()
            origin = lax.rem(my + N_DEV - 1 - h, N_DEV)
            out_ref[pl.ds(origin * m_per, m_per), :] = (
                jnp.dot(comm_ref[h], w_ref[...], preferred_element_type=jnp.float32)
                * scale
            )

    return pl.pallas_call(
        body,
        out_shape=jax.ShapeDtypeStruct((N_DEV * m_per, n_per), jnp.float32),
        in_specs=[
            pl.BlockSpec(memory_space=pltpu.VMEM),
            pl.BlockSpec(memory_space=pltpu.VMEM),
            pl.BlockSpec(memory_space=pltpu.SMEM),
            pl.BlockSpec(memory_space=pltpu.SMEM),
        ],
        out_specs=pl.BlockSpec(memory_space=pltpu.VMEM),
        scratch_shapes=[
            pltpu.VMEM((N_DEV - 1, m_per, k), jnp.float8_e5m2),
            pltpu.SemaphoreType.DMA((N_DEV - 1,)),
            pltpu.SemaphoreType.DMA((N_DEV - 1,)),
        ],
        compiler_params=pltpu.CompilerParams(
            collective_id=0,
            vmem_limit_bytes=100 * 1024 * 1024,
        ),
    )(x8, w8, scale_x, scale_w)


# device time: 142566 ns/iter; 1.6944x vs baseline; 1.6944x over previous
import jax
import jax.numpy as jnp
from jax import lax
from jax.experimental import pallas as pl
from jax.experimental.pallas import tpu as pltpu

N_DEV = 4


def kernel(x, w_mat, scale_x, scale_w):
    x8 = x.astype(jnp.float8_e5m2)
    w8 = w_mat.astype(jnp.float8_e5m2)
    m_per, k = x8.shape
    _, n_per = w8.shape
    H = m_per // 2
    Q = m_per // 4

    def body(x_ref, w_ref, sx_ref, sw_ref, out_ref,
             bufL, bufR, bufD, send_sems, recv_sems):
        my = lax.axis_index("i")
        left = lax.rem(my + N_DEV - 1, N_DEV)
        right = lax.rem(my + 1, N_DEV)
        oL = left
        oR = right
        oD = lax.rem(my + 2, N_DEV)

        barrier_sem = pltpu.get_barrier_semaphore()
        for nbr in (left, right):
            pl.semaphore_signal(
                barrier_sem, inc=1,
                device_id=(nbr,), device_id_type=pl.DeviceIdType.MESH,
            )
        pl.semaphore_wait(barrier_sem, 2)

        def copy(i, src, dst, dev):
            return pltpu.make_async_remote_copy(
                src_ref=src, dst_ref=dst,
                send_sem=send_sems.at[i], recv_sem=recv_sems.at[i],
                device_id=(dev,), device_id_type=pl.DeviceIdType.MESH,
            )

        a0 = copy(0, x_ref.at[pl.ds(0, H)], bufL.at[pl.ds(0, H)], right)
        a1 = copy(1, x_ref.at[pl.ds(H, H)], bufL.at[pl.ds(H, H)], right)
        c0 = copy(2, x_ref.at[pl.ds(0, H)], bufR.at[pl.ds(0, H)], left)
        c1 = copy(3, x_ref.at[pl.ds(H, H)], bufR.at[pl.ds(H, H)], left)
        a0.start(); a1.start(); c0.start(); c1.start()

        scale = sx_ref[0] * sw_ref[0]
        w = w_ref[...]

        def piece(buf_row0, nrows, origin, row_off):
            out_ref[pl.ds(origin * m_per + row_off, nrows), :] = (
                jnp.dot(buf_row0, w, preferred_element_type=jnp.float32)
                * scale
            )

        piece(x_ref[...], m_per, my, 0)

        a0.wait_recv()
        b0 = copy(4, bufL.at[pl.ds(0, Q)], bufD.at[pl.ds(0, Q)], right)
        b1 = copy(5, bufL.at[pl.ds(Q, Q)], bufD.at[pl.ds(Q, Q)], right)
        b0.start(); b1.start()
        piece(bufL[pl.ds(0, H), :], H, oL, 0)

        c0.wait_recv()
        piece(bufR[pl.ds(0, H), :], H, oR, 0)

        a1.wait_recv()
        piece(bufL[pl.ds(H, H), :], H, oL, H)

        c1.wait_recv()
        d0 = copy(6, bufR.at[pl.ds(H, Q)], bufD.at[pl.ds(H, Q)], left)
        d1 = copy(7, bufR.at[pl.ds(H + Q, Q)], bufD.at[pl.ds(H + Q, Q)], left)
        d0.start(); d1.start()
        piece(bufR[pl.ds(H, H), :], H, oR, H)

        b0.wait_recv()
        piece(bufD[pl.ds(0, Q), :], Q, oD, 0)
        d0.wait_recv()
        piece(bufD[pl.ds(H, Q), :], Q, oD, H)
        b1.wait_recv()
        piece(bufD[pl.ds(Q, Q), :], Q, oD, Q)
        d1.wait_recv()
        piece(bufD[pl.ds(H + Q, Q), :], Q, oD, H + Q)

        for r in (a0, a1, c0, c1, b0, b1, d0, d1):
            r.wait_send()

    return pl.pallas_call(
        body,
        out_shape=jax.ShapeDtypeStruct((N_DEV * m_per, n_per), jnp.float32),
        in_specs=[
            pl.BlockSpec(memory_space=pltpu.VMEM),
            pl.BlockSpec(memory_space=pltpu.VMEM),
            pl.BlockSpec(memory_space=pltpu.SMEM),
            pl.BlockSpec(memory_space=pltpu.SMEM),
        ],
        out_specs=pl.BlockSpec(memory_space=pltpu.VMEM),
        scratch_shapes=[
            pltpu.VMEM((m_per, k), jnp.float8_e5m2),
            pltpu.VMEM((m_per, k), jnp.float8_e5m2),
            pltpu.VMEM((m_per, k), jnp.float8_e5m2),
            pltpu.SemaphoreType.DMA((8,)),
            pltpu.SemaphoreType.DMA((8,)),
        ],
        compiler_params=pltpu.CompilerParams(
            collective_id=0,
            vmem_limit_bytes=100 * 1024 * 1024,
        ),
    )(x8, w8, scale_x, scale_w)


# device time: 106966 ns/iter; 2.2583x vs baseline; 1.3328x over previous
import jax
import jax.numpy as jnp
from jax import lax
from jax.experimental import pallas as pl
from jax.experimental.pallas import tpu as pltpu

N_DEV = 4
FP8 = jnp.float8_e5m2


def kernel(x, w_mat, scale_x, scale_w):
    m_per, k = x.shape
    _, n_per = w_mat.shape
    H = m_per // 2
    Q = m_per // 4
    KT = k // 4

    def body(x_hbm, w_hbm, sx_ref, sw_ref, out_hbm,
             x8, w8, bufL, bufR, bufD, x_stage, w_stage, acc,
             xsem, wsem, osem, send_sems, recv_sems):
        my = lax.axis_index("i")
        left = lax.rem(my + N_DEV - 1, N_DEV)
        right = lax.rem(my + 1, N_DEV)
        oD = lax.rem(my + 2, N_DEV)

        cp_x = []
        for i in range(2):
            c = pltpu.make_async_copy(
                x_hbm.at[pl.ds(i * H, H)], x_stage.at[i], xsem.at[i])
            c.start()
            cp_x.append(c)
        cp_w = []
        for i in range(2):
            c = pltpu.make_async_copy(
                w_hbm.at[pl.ds(i * KT, KT)], w_stage.at[i], wsem.at[i])
            c.start()
            cp_w.append(c)

        barrier_sem = pltpu.get_barrier_semaphore()
        for nbr in (left, right):
            pl.semaphore_signal(
                barrier_sem, inc=1,
                device_id=(nbr,), device_id_type=pl.DeviceIdType.MESH,
            )
        pl.semaphore_wait(barrier_sem, 2)

        def rcopy(i, src, dst, dev):
            return pltpu.make_async_remote_copy(
                src_ref=src, dst_ref=dst,
                send_sem=send_sems.at[i], recv_sem=recv_sems.at[i],
                device_id=(dev,), device_id_type=pl.DeviceIdType.MESH,
            )

        cp_x[0].wait()
        x8[pl.ds(0, H), :] = x_stage[0].astype(FP8)
        a0 = rcopy(0, x8.at[pl.ds(0, H)], bufL.at[pl.ds(0, H)], right)
        c0 = rcopy(2, x8.at[pl.ds(0, H)], bufR.at[pl.ds(0, H)], left)
        a0.start(); c0.start()

        cp_x[1].wait()
        x8[pl.ds(H, H), :] = x_stage[1].astype(FP8)
        a1 = rcopy(1, x8.at[pl.ds(H, H)], bufL.at[pl.ds(H, H)], right)
        c1 = rcopy(3, x8.at[pl.ds(H, H)], bufR.at[pl.ds(H, H)], left)
        a1.start(); c1.start()

        for i in range(4):
            cp_w[i].wait()
            w8[pl.ds(i * KT, KT), :] = w_stage[i % 2].astype(FP8)
            if i + 2 < 4:
                c = pltpu.make_async_copy(
                    w_hbm.at[pl.ds((i + 2) * KT, KT)],
                    w_stage.at[i % 2], wsem.at[i % 2])
                c.start()
                cp_w.append(c)

        scale = sx_ref[0] * sw_ref[0]
        w = w8[...]

        out_cp = [None, None]
        n_pieces = [0]

        def piece(src_ref, src_off, origin, row_off):
            slot = n_pieces[0] % 2
            if out_cp[slot] is not None:
                out_cp[slot].wait()
            acc[slot] = (
                jnp.dot(src_ref[pl.ds(src_off, Q), :], w,
                        preferred_element_type=jnp.float32)
                * scale
            )
            c = pltpu.make_async_copy(
                acc.at[slot],
                out_hbm.at[pl.ds(origin * m_per + row_off, Q)],
                osem.at[slot],
            )
            c.start()
            out_cp[slot] = c
            n_pieces[0] += 1

        for q in range(4):
            piece(x8, q * Q, my, q * Q)

        a0.wait_recv()
        b0 = rcopy(4, bufL.at[pl.ds(0, Q)], bufD.at[pl.ds(0, Q)], right)
        b1 = rcopy(5, bufL.at[pl.ds(Q, Q)], bufD.at[pl.ds(Q, Q)], right)
        b0.start(); b1.start()
        piece(bufL, 0, left, 0)
        piece(bufL, Q, left, Q)

        c0.wait_recv()
        piece(bufR, 0, right, 0)
        piece(bufR, Q, right, Q)

        a1.wait_recv()
        piece(bufL, H, left, H)
        piece(bufL, H + Q, left, H + Q)

        c1.wait_recv()
        d0 = rcopy(6, bufR.at[pl.ds(H, Q)], bufD.at[pl.ds(H, Q)], left)
        d1 = rcopy(7, bufR.at[pl.ds(H + Q, Q)], bufD.at[pl.ds(H + Q, Q)], left)
        d0.start(); d1.start()
        piece(bufR, H, right, H)
        piece(bufR, H + Q, right, H + Q)

        b0.wait_recv()
        piece(bufD, 0, oD, 0)
        d0.wait_recv()
        piece(bufD, H, oD, H)
        b1.wait_recv()
        piece(bufD, Q, oD, Q)
        d1.wait_recv()
        piece(bufD, H + Q, oD, H + Q)

        for c in out_cp:
            c.wait()
        for r in (a0, a1, c0, c1, b0, b1, d0, d1):
            r.wait_send()

    return pl.pallas_call(
        body,
        out_shape=jax.ShapeDtypeStruct((N_DEV * m_per, n_per), jnp.float32),
        in_specs=[
            pl.BlockSpec(memory_space=pl.ANY),
            pl.BlockSpec(memory_space=pl.ANY),
            pl.BlockSpec(memory_space=pltpu.SMEM),
            pl.BlockSpec(memory_space=pltpu.SMEM),
        ],
        out_specs=pl.BlockSpec(memory_space=pl.ANY),
        scratch_shapes=[
            pltpu.VMEM((m_per, k), FP8),
            pltpu.VMEM((k, n_per), FP8),
            pltpu.VMEM((m_per, k), FP8),
            pltpu.VMEM((m_per, k), FP8),
            pltpu.VMEM((m_per, k), FP8),
            pltpu.VMEM((2, H, k), jnp.float32),
            pltpu.VMEM((2, KT, n_per), jnp.float32),
            pltpu.VMEM((2, Q, n_per), jnp.float32),
            pltpu.SemaphoreType.DMA((2,)),
            pltpu.SemaphoreType.DMA((2,)),
            pltpu.SemaphoreType.DMA((2,)),
            pltpu.SemaphoreType.DMA((8,)),
            pltpu.SemaphoreType.DMA((8,)),
        ],
        compiler_params=pltpu.CompilerParams(
            collective_id=0,
            vmem_limit_bytes=100 * 1024 * 1024,
        ),
    )(x, w_mat, scale_x, scale_w)


# device time: 104305 ns/iter; 2.3159x vs baseline; 1.0255x over previous
import jax
import jax.numpy as jnp
from jax import lax
from jax.experimental import pallas as pl
from jax.experimental.pallas import tpu as pltpu

N_DEV = 4
FP8 = jnp.float8_e5m2


def kernel(x, w_mat, scale_x, scale_w):
    m_per, k = x.shape
    _, n_per = w_mat.shape
    H = m_per // 2
    Q = m_per // 4
    KT = k // 4

    def body(x_hbm, w_hbm, sx_ref, sw_ref, out_hbm,
             x8, w8, bufL, bufR, bufD, x_stage, w_stage, acc,
             xsem, wsem, osem, send_sems, recv_sems):
        my = lax.axis_index("i")
        left = lax.rem(my + N_DEV - 1, N_DEV)
        right = lax.rem(my + 1, N_DEV)
        oD = lax.rem(my + 2, N_DEV)

        cp_x0 = pltpu.make_async_copy(
            x_hbm.at[pl.ds(0, H)], x_stage.at[0], xsem.at[0])
        cp_x0.start()

        barrier_sem = pltpu.get_barrier_semaphore()
        for nbr in (left, right):
            pl.semaphore_signal(
                barrier_sem, inc=1,
                device_id=(nbr,), device_id_type=pl.DeviceIdType.MESH,
            )
        pl.semaphore_wait(barrier_sem, 2)

        def rcopy(i, src, dst, dev):
            return pltpu.make_async_remote_copy(
                src_ref=src, dst_ref=dst,
                send_sem=send_sems.at[i], recv_sem=recv_sems.at[i],
                device_id=(dev,), device_id_type=pl.DeviceIdType.MESH,
            )

        cp_x0.wait()
        x8[pl.ds(0, H), :] = x_stage[0].astype(FP8)
        a0 = rcopy(0, x8.at[pl.ds(0, H)], bufL.at[pl.ds(0, H)], right)
        c0 = rcopy(2, x8.at[pl.ds(0, H)], bufR.at[pl.ds(0, H)], left)
        a0.start(); c0.start()

        cp_x1 = pltpu.make_async_copy(
            x_hbm.at[pl.ds(H, H)], x_stage.at[1], xsem.at[1])
        cp_x1.start()
        cp_w = []
        for i in range(2):
            c = pltpu.make_async_copy(
                w_hbm.at[pl.ds(i * KT, KT)], w_stage.at[i], wsem.at[i])
            c.start()
            cp_w.append(c)

        cp_x1.wait()
        x8[pl.ds(H, H), :] = x_stage[1].astype(FP8)
        a1 = rcopy(1, x8.at[pl.ds(H, H)], bufL.at[pl.ds(H, H)], right)
        c1 = rcopy(3, x8.at[pl.ds(H, H)], bufR.at[pl.ds(H, H)], left)
        a1.start(); c1.start()

        for i in range(4):
            cp_w[i].wait()
            w8[pl.ds(i * KT, KT), :] = w_stage[i % 2].astype(FP8)
            if i + 2 < 4:
                c = pltpu.make_async_copy(
                    w_hbm.at[pl.ds((i + 2) * KT, KT)],
                    w_stage.at[i % 2], wsem.at[i % 2])
                c.start()
                cp_w.append(c)

        scale = sx_ref[0] * sw_ref[0]
        w = w8[...]

        out_cp = [None, None]
        n_pieces = [0]

        def piece(src_ref, src_off, origin, row_off):
            slot = n_pieces[0] % 2
            if out_cp[slot] is not None:
                out_cp[slot].wait()
            acc[slot] = (
                jnp.dot(src_ref[pl.ds(src_off, Q), :], w,
                        preferred_element_type=jnp.float32)
                * scale
            )
            c = pltpu.make_async_copy(
                acc.at[slot],
                out_hbm.at[pl.ds(origin * m_per + row_off, Q)],
                osem.at[slot],
            )
            c.start()
            out_cp[slot] = c
            n_pieces[0] += 1

        for q in range(4):
            piece(x8, q * Q, my, q * Q)

        a0.wait_recv()
        b0 = rcopy(4, bufL.at[pl.ds(0, Q)], bufD.at[pl.ds(0, Q)], right)
        b1 = rcopy(5, bufL.at[pl.ds(Q, Q)], bufD.at[pl.ds(Q, Q)], right)
        b0.start(); b1.start()
        piece(bufL, 0, left, 0)
        piece(bufL, Q, left, Q)

        c0.wait_recv()
        piece(bufR, 0, right, 0)
        piece(bufR, Q, right, Q)

        c1.wait_recv()
        d0 = rcopy(6, bufR.at[pl.ds(H, Q)], bufD.at[pl.ds(H, Q)], left)
        d1 = rcopy(7, bufR.at[pl.ds(H + Q, Q)], bufD.at[pl.ds(H + Q, Q)], left)
        d0.start(); d1.start()
        a1.wait_recv()
        piece(bufL, H, left, H)
        piece(bufL, H + Q, left, H + Q)
        piece(bufR, H, right, H)
        piece(bufR, H + Q, right, H + Q)

        b0.wait_recv()
        piece(bufD, 0, oD, 0)
        d0.wait_recv()
        piece(bufD, H, oD, H)
        b1.wait_recv()
        piece(bufD, Q, oD, Q)
        d1.wait_recv()
        piece(bufD, H + Q, oD, H + Q)

        for c in out_cp:
            c.wait()
        for r in (a0, a1, c0, c1, b0, b1, d0, d1):
            r.wait_send()

    return pl.pallas_call(
        body,
        out_shape=jax.ShapeDtypeStruct((N_DEV * m_per, n_per), jnp.float32),
        in_specs=[
            pl.BlockSpec(memory_space=pl.ANY),
            pl.BlockSpec(memory_space=pl.ANY),
            pl.BlockSpec(memory_space=pltpu.SMEM),
            pl.BlockSpec(memory_space=pltpu.SMEM),
        ],
        out_specs=pl.BlockSpec(memory_space=pl.ANY),
        scratch_shapes=[
            pltpu.VMEM((m_per, k), FP8),
            pltpu.VMEM((k, n_per), FP8),
            pltpu.VMEM((m_per, k), FP8),
            pltpu.VMEM((m_per, k), FP8),
            pltpu.VMEM((m_per, k), FP8),
            pltpu.VMEM((2, H, k), jnp.float32),
            pltpu.VMEM((2, KT, n_per), jnp.float32),
            pltpu.VMEM((2, Q, n_per), jnp.float32),
            pltpu.SemaphoreType.DMA((2,)),
            pltpu.SemaphoreType.DMA((2,)),
            pltpu.SemaphoreType.DMA((2,)),
            pltpu.SemaphoreType.DMA((8,)),
            pltpu.SemaphoreType.DMA((8,)),
        ],
        compiler_params=pltpu.CompilerParams(
            collective_id=0,
            vmem_limit_bytes=100 * 1024 * 1024,
        ),
    )(x, w_mat, scale_x, scale_w)


# device time: 101023 ns/iter; 2.3912x vs baseline; 1.0325x over previous
import jax
import jax.numpy as jnp
from jax import lax
from jax.experimental import pallas as pl
from jax.experimental.pallas import tpu as pltpu

N_DEV = 4
FP8 = jnp.float8_e5m2


def kernel(x, w_mat, scale_x, scale_w):
    m_per, k = x.shape
    _, n_per = w_mat.shape
    H = m_per // 2
    Q = m_per // 4
    E = m_per // 8
    KT = k // 4

    def body(x_hbm, w_hbm, sx_ref, sw_ref, out_hbm,
             x8, w8, bufL, bufR, bufD, x_stage, w_stage, acc,
             xsem, wsem, osem, send_sems, recv_sems):
        my = lax.axis_index("i")
        left = lax.rem(my + N_DEV - 1, N_DEV)
        right = lax.rem(my + 1, N_DEV)
        oD = lax.rem(my + 2, N_DEV)

        cp_q0 = pltpu.make_async_copy(
            x_hbm.at[pl.ds(0, Q)], x_stage.at[0, pl.ds(0, Q)], xsem.at[0])
        cp_q0.start()

        barrier_sem = pltpu.get_barrier_semaphore()
        for nbr in (left, right):
            pl.semaphore_signal(
                barrier_sem, inc=1,
                device_id=(nbr,), device_id_type=pl.DeviceIdType.MESH,
            )
        pl.semaphore_wait(barrier_sem, 2)

        def rcopy(i, src, dst, dev):
            return pltpu.make_async_remote_copy(
                src_ref=src, dst_ref=dst,
                send_sem=send_sems.at[i], recv_sem=recv_sems.at[i],
                device_id=(dev,), device_id_type=pl.DeviceIdType.MESH,
            )

        cp_q0.wait()
        x8[pl.ds(0, Q), :] = x_stage[0, pl.ds(0, Q), :].astype(FP8)
        a0a = rcopy(0, x8.at[pl.ds(0, Q)], bufL.at[pl.ds(0, Q)], right)
        c0a = rcopy(2, x8.at[pl.ds(0, Q)], bufR.at[pl.ds(0, Q)], left)
        a0a.start(); c0a.start()

        cp_q1 = pltpu.make_async_copy(
            x_hbm.at[pl.ds(Q, Q)], x_stage.at[0, pl.ds(Q, Q)], xsem.at[1])
        cp_q1.start()
        cp_q1.wait()
        x8[pl.ds(Q, Q), :] = x_stage[0, pl.ds(Q, Q), :].astype(FP8)
        a0b = rcopy(8, x8.at[pl.ds(Q, Q)], bufL.at[pl.ds(Q, Q)], right)
        c0b = rcopy(9, x8.at[pl.ds(Q, Q)], bufR.at[pl.ds(Q, Q)], left)
        a0b.start(); c0b.start()

        cp_x1 = pltpu.make_async_copy(
            x_hbm.at[pl.ds(H, H)], x_stage.at[1], xsem.at[2])
        cp_x1.start()
        cp_w = []
        for i in range(2):
            c = pltpu.make_async_copy(
                w_hbm.at[pl.ds(i * KT, KT)], w_stage.at[i], wsem.at[i])
            c.start()
            cp_w.append(c)

        cp_x1.wait()
        x8[pl.ds(H, H), :] = x_stage[1].astype(FP8)
        a1 = rcopy(1, x8.at[pl.ds(H, H)], bufL.at[pl.ds(H, H)], right)
        c1 = rcopy(3, x8.at[pl.ds(H, H)], bufR.at[pl.ds(H, H)], left)
        a1.start(); c1.start()

        for i in range(4):
            cp_w[i].wait()
            w8[pl.ds(i * KT, KT), :] = w_stage[i % 2].astype(FP8)
            if i + 2 < 4:
                c = pltpu.make_async_copy(
                    w_hbm.at[pl.ds((i + 2) * KT, KT)],
                    w_stage.at[i % 2], wsem.at[i % 2])
                c.start()
                cp_w.append(c)

        scale = sx_ref[0] * sw_ref[0]
        w = w8[...]

        out_cp = [None, None]
        n_pieces = [0]

        def piece(src_ref, src_off, nr, origin, row_off):
            slot = n_pieces[0] % 2
            if out_cp[slot] is not None:
                out_cp[slot].wait()
            acc[slot, pl.ds(0, nr), :] = (
                jnp.dot(src_ref[pl.ds(src_off, nr), :], w,
                        preferred_element_type=jnp.float32)
                * scale
            )
            c = pltpu.make_async_copy(
                acc.at[slot, pl.ds(0, nr)],
                out_hbm.at[pl.ds(origin * m_per + row_off, nr)],
                osem.at[slot],
            )
            c.start()
            out_cp[slot] = c
            n_pieces[0] += 1

        for q in range(4):
            piece(x8, q * Q, Q, my, q * Q)

        a0a.wait_recv()
        b0 = rcopy(4, bufL.at[pl.ds(0, Q)], bufD.at[pl.ds(0, Q)], right)
        b0.start()
        a0b.wait_recv()
        b1a = rcopy(5, bufL.at[pl.ds(Q, E)], bufD.at[pl.ds(Q, E)], right)
        b1b = rcopy(10, bufL.at[pl.ds(Q + E, E)], bufD.at[pl.ds(Q + E, E)], right)
        b1a.start(); b1b.start()
        piece(bufL, 0, Q, left, 0)
        c0a.wait_recv()
        c0b.wait_recv()
        piece(bufL, Q, Q, left, Q)
        piece(bufR, 0, Q, right, 0)
        piece(bufR, Q, Q, right, Q)

        c1.wait_recv()
        d0 = rcopy(6, bufR.at[pl.ds(H, Q)], bufD.at[pl.ds(H, Q)], left)
        d1a = rcopy(7, bufR.at[pl.ds(H + Q, E)], bufD.at[pl.ds(H + Q, E)], left)
        d1b = rcopy(11, bufR.at[pl.ds(H + Q + E, E)],
                    bufD.at[pl.ds(H + Q + E, E)], left)
        d0.start(); d1a.start(); d1b.start()
        a1.wait_recv()
        piece(bufL, H, Q, left, H)
        piece(bufL, H + Q, Q, left, H + Q)
        piece(bufR, H, Q, right, H)
        piece(bufR, H + Q, Q, right, H + Q)

        b0.wait_recv()
        piece(bufD, 0, Q, oD, 0)
        d0.wait_recv()
        piece(bufD, H, Q, oD, H)
        b1a.wait_recv()
        piece(bufD, Q, E, oD, Q)
        d1a.wait_recv()
        piece(bufD, H + Q, E, oD, H + Q)
        b1b.wait_recv()
        piece(bufD, Q + E, E, oD, Q + E)
        d1b.wait_recv()
        piece(bufD, H + Q + E, E, oD, H + Q + E)

        for c in out_cp:
            c.wait()
        for r in (a0a, a0b, a1, c0a, c0b, c1, b0, b1a, b1b, d0, d1a, d1b):
            r.wait_send()

    return pl.pallas_call(
        body,
        out_shape=jax.ShapeDtypeStruct((N_DEV * m_per, n_per), jnp.float32),
        in_specs=[
            pl.BlockSpec(memory_space=pl.ANY),
            pl.BlockSpec(memory_space=pl.ANY),
            pl.BlockSpec(memory_space=pltpu.SMEM),
            pl.BlockSpec(memory_space=pltpu.SMEM),
        ],
        out_specs=pl.BlockSpec(memory_space=pl.ANY),
        scratch_shapes=[
            pltpu.VMEM((m_per, k), FP8),
            pltpu.VMEM((k, n_per), FP8),
            pltpu.VMEM((m_per, k), FP8),
            pltpu.VMEM((m_per, k), FP8),
            pltpu.VMEM((m_per, k), FP8),
            pltpu.VMEM((2, H, k), jnp.float32),
            pltpu.VMEM((2, KT, n_per), jnp.float32),
            pltpu.VMEM((2, Q, n_per), jnp.float32),
            pltpu.SemaphoreType.DMA((3,)),
            pltpu.SemaphoreType.DMA((2,)),
            pltpu.SemaphoreType.DMA((2,)),
            pltpu.SemaphoreType.DMA((12,)),
            pltpu.SemaphoreType.DMA((12,)),
        ],
        compiler_params=pltpu.CompilerParams(
            collective_id=0,
            vmem_limit_bytes=100 * 1024 * 1024,
        ),
    )(x, w_mat, scale_x, scale_w)
